# baseline (speedup 1.0000x reference)
import jax
import jax.numpy as jnp
from jax.experimental import pallas as pl

_P = 1024
_KN = 64
_R1, _R2 = 0.2, 0.4
_EPS = 1e-5
_BLK = 4096


def _matmul_sum_kernel(x_ref, w_ref, m_ref, z_ref, acc_ref):
    z = jnp.dot(x_ref[...], w_ref[...], preferred_element_type=jnp.float32)
    z_ref[...] = z

    @pl.when(pl.program_id(0) == 0)
    def _():
        acc_ref[...] = jnp.zeros_like(acc_ref)

    acc_ref[...] += jnp.sum(z * m_ref[...], axis=0, keepdims=True)


def _var_kernel(z_ref, m_ref, mu_ref, acc_ref):
    d = z_ref[...] - mu_ref[...]

    @pl.when(pl.program_id(0) == 0)
    def _():
        acc_ref[...] = jnp.zeros_like(acc_ref)

    acc_ref[...] += jnp.sum(d * d * m_ref[...], axis=0, keepdims=True)


def _norm_relu_kernel(z_ref, mu_ref, sd_ref, h_ref):
    h_ref[...] = jnp.maximum((z_ref[...] - mu_ref[...]) / sd_ref[...], 0.0)


def _linear_bias_kernel(x_ref, w_ref, b_ref, o_ref):
    o_ref[...] = jnp.dot(x_ref[...], w_ref[...],
                         preferred_element_type=jnp.float32) + b_ref[...]


def _pl_matmul_sum(x, w, mask):
    n, ci = x.shape
    co = w.shape[1]
    blk = min(n, _BLK)
    grid = n // blk
    z, acc = pl.pallas_call(
        _matmul_sum_kernel,
        grid=(grid,),
        in_specs=[
            pl.BlockSpec((blk, ci), lambda i: (i, 0)),
            pl.BlockSpec((ci, co), lambda i: (0, 0)),
            pl.BlockSpec((blk, 1), lambda i: (i, 0)),
        ],
        out_specs=[
            pl.BlockSpec((blk, co), lambda i: (i, 0)),
            pl.BlockSpec((1, co), lambda i: (0, 0)),
        ],
        out_shape=[
            jax.ShapeDtypeStruct((n, co), jnp.float32),
            jax.ShapeDtypeStruct((1, co), jnp.float32),
        ],
    )(x, w, mask)
    return z, acc


def _pl_var(z, mask, mu):
    n, co = z.shape
    blk = min(n, _BLK)
    grid = n // blk
    return pl.pallas_call(
        _var_kernel,
        grid=(grid,),
        in_specs=[
            pl.BlockSpec((blk, co), lambda i: (i, 0)),
            pl.BlockSpec((blk, 1), lambda i: (i, 0)),
            pl.BlockSpec((1, co), lambda i: (0, 0)),
        ],
        out_specs=pl.BlockSpec((1, co), lambda i: (0, 0)),
        out_shape=jax.ShapeDtypeStruct((1, co), jnp.float32),
    )(z, mask, mu)


def _pl_norm_relu(z, mu, sd):
    n, co = z.shape
    blk = min(n, _BLK)
    grid = n // blk
    return pl.pallas_call(
        _norm_relu_kernel,
        grid=(grid,),
        in_specs=[
            pl.BlockSpec((blk, co), lambda i: (i, 0)),
            pl.BlockSpec((1, co), lambda i: (0, 0)),
            pl.BlockSpec((1, co), lambda i: (0, 0)),
        ],
        out_specs=pl.BlockSpec((blk, co), lambda i: (i, 0)),
        out_shape=jax.ShapeDtypeStruct((n, co), jnp.float32),
    )(z, mu, sd)


def _pl_linear_bias(x, w, b):
    n, ci = x.shape
    co = w.shape[1]
    blk = min(n, _BLK)
    grid = n // blk
    return pl.pallas_call(
        _linear_bias_kernel,
        grid=(grid,),
        in_specs=[
            pl.BlockSpec((blk, ci), lambda i: (i, 0)),
            pl.BlockSpec((ci, co), lambda i: (0, 0)),
            pl.BlockSpec((1, co), lambda i: (0, 0)),
        ],
        out_specs=pl.BlockSpec((blk, co), lambda i: (i, 0)),
        out_shape=jax.ShapeDtypeStruct((n, co), jnp.float32),
    )(x, w, b.reshape(1, co))


def _mlp_bn(h, ws, mask):
    m2 = mask[:, None]
    denom = jnp.sum(mask)
    for w in ws:
        z, s1 = _pl_matmul_sum(h, w, m2)
        mu = s1 / denom
        var = _pl_var(z, m2, mu) / denom
        sd = jnp.sqrt(var + _EPS)
        h = _pl_norm_relu(z, mu, sd)
    return h


def _fps(pos, m):
    d0 = jnp.sum((pos - pos[0]) ** 2, axis=1)

    def step(carry, _):
        dists, last = carry
        d = jnp.sum((pos - pos[last]) ** 2, axis=1)
        dists = jnp.minimum(dists, d)
        nxt = jnp.argmax(dists).astype(jnp.int32)
        return (dists, nxt), nxt

    _, rest = jax.lax.scan(step, (d0, jnp.int32(0)), None, length=m - 1)
    return jnp.concatenate([jnp.zeros((1,), jnp.int32), rest])


def _radius_knn(pos_src, pos_dst, r):
    d2 = jnp.sum((pos_dst[:, None, :] - pos_src[None, :, :]) ** 2, axis=-1)
    scored = jnp.where(d2 <= r * r, -d2, -1e30)
    vals, idx = jax.lax.top_k(scored, _KN)
    return idx, vals > -1e20


def _set_abstraction(x, pos, ratio, r, ws):
    sg = jax.lax.stop_gradient
    m = int(pos.shape[1] * ratio)
    sel = jax.vmap(lambda p: _fps(p, m))(sg(pos))
    pos_dst = jnp.take_along_axis(pos, sel[:, :, None], axis=1)
    idx, valid = jax.vmap(lambda ps, pd: _radius_knn(ps, pd, r))(sg(pos), sg(pos_dst))
    pos_j = jax.vmap(lambda p, i: p[i])(pos, idx)
    rel = pos_j - pos_dst[:, :, None, :]
    if x is None:
        msg = rel
    else:
        x_j = jax.vmap(lambda xx, i: xx[i])(x, idx)
        msg = jnp.concatenate([x_j, rel], axis=-1)
    bn, mn, kn, cn = msg.shape
    h = _mlp_bn(msg.reshape(bn * mn * kn, cn), ws,
                valid.reshape(-1).astype(jnp.float32))
    h = h.reshape(bn, mn, kn, -1)
    h = jnp.where(valid[..., None], h, -1e30)
    return jnp.max(h, axis=2), pos_dst


def _knn_interpolate(x_c, pos_c, pos_f, k=3):
    def one(xc, pc, pf):
        d2 = jnp.sum((pf[:, None, :] - pc[None, :, :]) ** 2, axis=-1)
        neg, idx = jax.lax.top_k(-d2, k)
        w = 1.0 / jnp.maximum(-neg, 1e-16)
        xk = xc[idx]
        return jnp.sum(w[:, :, None] * xk, axis=1) / jnp.sum(w, axis=1, keepdims=True)

    return jax.vmap(one)(x_c, pos_c, pos_f)


def kernel(pos, batch, W1a, W1b, W1c, W2a, W2b, W2c, Wf2a, Wf2b, Wf1a, Wf1b,
           Wr1, Wr2, br2, Wc1, Wc2, bc2):
    b = pos.shape[0] // _P
    m1, m2 = _P // 2, _P // 8
    posb = pos.reshape(b, _P, 3)
    x1, pos1 = _set_abstraction(None, posb, 0.5, _R1, [W1a, W1b, W1c])
    x2, pos2 = _set_abstraction(x1, pos1, 0.25, _R2, [W2a, W2b, W2c])
    summary_pos = pos2.reshape(b * m2, 3)
    xi = _knn_interpolate(x2, pos2, pos1)
    h = jnp.concatenate([xi, x1], axis=-1).reshape(b * m1, 384)
    ones1 = jnp.ones((b * m1,), jnp.float32)
    x1_up = _mlp_bn(h, [Wf2a, Wf2b], ones1).reshape(b, m1, 128)
    xi0 = _knn_interpolate(x1_up, pos1, posb).reshape(b * _P, 128)
    ones0 = jnp.ones((b * _P,), jnp.float32)
    x0_up = _mlp_bn(xi0, [Wf1a, Wf1b], ones0)
    rh = _mlp_bn(x0_up, [Wr1], ones0)
    recon_pos = _pl_linear_bias(rh, Wr2, br2)
    g = jnp.max(x2, axis=1)
    ch = _mlp_bn(g, [Wc1], jnp.ones((b,), jnp.float32))
    logits = _pl_linear_bias(ch, Wc2, bc2)
    return summary_pos, recon_pos, logits
